# baseline (speedup 1.0000x reference)
import jax
import jax.numpy as jnp
from jax.experimental import pallas as pl

N_EXPERTS = 8
TOP_K = 2
BLOCK_T = 4096


def _gate_kernel(x_ref, w_ref, probs_ref, tv_ref, ti_ref):
    x = x_ref[...]
    w = w_ref[...]
    scores = jax.lax.dot_general(
        w, x, (((1,), (1,)), ((), ())), preferred_element_type=jnp.float32
    )
    m = jnp.max(scores, axis=0, keepdims=True)
    e = jnp.exp(scores - m)
    s = jnp.sum(e, axis=0, keepdims=True)
    probs = e / s

    v1 = jnp.max(probs, axis=0, keepdims=True)
    i1 = jnp.argmax(probs, axis=0).reshape(1, -1)
    row = jax.lax.broadcasted_iota(jnp.int32, probs.shape, 0)
    masked = jnp.where(row == i1, -jnp.inf, probs)
    v2 = jnp.max(masked, axis=0, keepdims=True)
    i2 = jnp.argmax(masked, axis=0).reshape(1, -1)
    denom = v1 + v2 + 1e-9
    tv = jnp.concatenate([v1 / denom, v2 / denom], axis=0)
    ti = jnp.concatenate([i1, i2], axis=0)
    probs_ref[...] = jnp.transpose(probs)
    tv_ref[...] = jnp.transpose(tv)
    ti_ref[...] = jnp.transpose(ti).astype(jnp.int32)


def kernel(x, weight):
    n_tok, dim = x.shape
    n_exp = weight.shape[0]
    grid = (n_tok // BLOCK_T,)
    probs_t, tv_t, ti_t = pl.pallas_call(
        _gate_kernel,
        grid=grid,
        in_specs=[
            pl.BlockSpec((BLOCK_T, dim), lambda i: (i, 0)),
            pl.BlockSpec((n_exp, dim), lambda i: (0, 0)),
        ],
        out_specs=[
            pl.BlockSpec((BLOCK_T, n_exp), lambda i: (i, 0)),
            pl.BlockSpec((BLOCK_T, TOP_K), lambda i: (i, 0)),
            pl.BlockSpec((BLOCK_T, TOP_K), lambda i: (i, 0)),
        ],
        out_shape=[
            jax.ShapeDtypeStruct((n_tok, n_exp), jnp.float32),
            jax.ShapeDtypeStruct((n_tok, TOP_K), jnp.float32),
            jax.ShapeDtypeStruct((n_tok, TOP_K), jnp.int32),
        ],
    )(x, weight)
    return tv_t, ti_t, probs_t


# speedup vs baseline: 2.0810x; 2.0810x over previous
import jax
import jax.numpy as jnp
from jax.experimental import pallas as pl
from jax.experimental.pallas import tpu as pltpu

N_EXPERTS = 8
TOP_K = 2
BLOCK_T = 4096
D_SPLIT = 2


def _gate_kernel(x_ref, w_ref, probs_ref, tv_ref, ti_ref, acc_ref):
    j = pl.program_id(1)
    partial = jax.lax.dot_general(
        w_ref[...], x_ref[...], (((1,), (1,)), ((), ())),
        preferred_element_type=jnp.float32,
    )

    @pl.when(j == 0)
    def _():
        acc_ref[...] = partial

    @pl.when(j > 0)
    def _():
        acc_ref[...] += partial

    @pl.when(j == D_SPLIT - 1)
    def _():
        scores = acc_ref[...]
        m = jnp.max(scores, axis=0, keepdims=True)
        e = jnp.exp(scores - m)
        s = jnp.sum(e, axis=0, keepdims=True)
        probs = e / s
        probs_ref[...] = probs

        v1 = jnp.max(probs, axis=0, keepdims=True)
        i1 = jnp.argmax(probs, axis=0).reshape(1, -1)
        row = jax.lax.broadcasted_iota(jnp.int32, probs.shape, 0)
        masked = jnp.where(row == i1, -jnp.inf, probs)
        v2 = jnp.max(masked, axis=0, keepdims=True)
        i2 = jnp.argmax(masked, axis=0).reshape(1, -1)
        denom = v1 + v2 + 1e-9
        tv_ref[...] = jnp.concatenate([v1 / denom, v2 / denom], axis=0)
        ti_ref[...] = jnp.concatenate([i1, i2], axis=0).astype(jnp.int32)


def kernel(x, weight):
    n_tok, dim = x.shape
    n_exp = weight.shape[0]
    d_blk = dim // D_SPLIT
    grid = (n_tok // BLOCK_T, D_SPLIT)
    probs_t, tv_t, ti_t = pl.pallas_call(
        _gate_kernel,
        grid=grid,
        in_specs=[
            pl.BlockSpec((BLOCK_T, d_blk), lambda i, j: (i, j)),
            pl.BlockSpec((n_exp, d_blk), lambda i, j: (0, j)),
        ],
        out_specs=[
            pl.BlockSpec((n_exp, BLOCK_T), lambda i, j: (0, i)),
            pl.BlockSpec((TOP_K, BLOCK_T), lambda i, j: (0, i)),
            pl.BlockSpec((TOP_K, BLOCK_T), lambda i, j: (0, i)),
        ],
        out_shape=[
            jax.ShapeDtypeStruct((n_exp, n_tok), jnp.float32),
            jax.ShapeDtypeStruct((TOP_K, n_tok), jnp.float32),
            jax.ShapeDtypeStruct((TOP_K, n_tok), jnp.int32),
        ],
        scratch_shapes=[pltpu.VMEM((n_exp, BLOCK_T), jnp.float32)],
    )(x, weight)
    return tv_t.T, ti_t.T, probs_t.T


# speedup vs baseline: 2.2098x; 1.0619x over previous
import jax
import jax.numpy as jnp
from jax.experimental import pallas as pl

N_EXPERTS = 8
TOP_K = 2
BLOCK_T = 4096


def _gate_kernel(xa_ref, xb_ref, w_ref, probs_ref, tv_ref, ti_ref):
    w = w_ref[...]
    d2 = xa_ref.shape[1]
    scores = jax.lax.dot_general(
        w[:, :d2], xa_ref[...], (((1,), (1,)), ((), ())),
        preferred_element_type=jnp.float32,
    ) + jax.lax.dot_general(
        w[:, d2:], xb_ref[...], (((1,), (1,)), ((), ())),
        preferred_element_type=jnp.float32,
    )
    m = jnp.max(scores, axis=0, keepdims=True)
    e = jnp.exp(scores - m)
    s = jnp.sum(e, axis=0, keepdims=True)
    probs = e / s
    probs_ref[...] = probs

    v1 = jnp.max(probs, axis=0, keepdims=True)
    i1 = jnp.argmax(probs, axis=0).reshape(1, -1)
    row = jax.lax.broadcasted_iota(jnp.int32, probs.shape, 0)
    masked = jnp.where(row == i1, -jnp.inf, probs)
    v2 = jnp.max(masked, axis=0, keepdims=True)
    i2 = jnp.argmax(masked, axis=0).reshape(1, -1)
    denom = v1 + v2 + 1e-9
    tv_ref[...] = jnp.concatenate([v1 / denom, v2 / denom], axis=0)
    ti_ref[...] = jnp.concatenate([i1, i2], axis=0).astype(jnp.int32)


def kernel(x, weight):
    n_tok, dim = x.shape
    n_exp = weight.shape[0]
    d2 = dim // 2
    grid = (n_tok // BLOCK_T,)
    probs_t, tv_t, ti_t = pl.pallas_call(
        _gate_kernel,
        grid=grid,
        in_specs=[
            pl.BlockSpec((BLOCK_T, d2), lambda i: (i, 0)),
            pl.BlockSpec((BLOCK_T, d2), lambda i: (i, 1)),
            pl.BlockSpec((n_exp, dim), lambda i: (0, 0)),
        ],
        out_specs=[
            pl.BlockSpec((n_exp, BLOCK_T), lambda i: (0, i)),
            pl.BlockSpec((TOP_K, BLOCK_T), lambda i: (0, i)),
            pl.BlockSpec((TOP_K, BLOCK_T), lambda i: (0, i)),
        ],
        out_shape=[
            jax.ShapeDtypeStruct((n_exp, n_tok), jnp.float32),
            jax.ShapeDtypeStruct((TOP_K, n_tok), jnp.float32),
            jax.ShapeDtypeStruct((TOP_K, n_tok), jnp.int32),
        ],
    )(x, x, weight)
    return tv_t.T, ti_t.T, probs_t.T


# speedup vs baseline: 2.2651x; 1.0251x over previous
import jax
import jax.numpy as jnp
from jax.experimental import pallas as pl

N_EXPERTS = 8
TOP_K = 2
BLOCK_T = 4096


def _gate_kernel(xa_ref, xb_ref, w_ref, probs_ref, tv_ref, ti_ref):
    w = w_ref[...]
    scores = jnp.concatenate([
        jax.lax.dot_general(
            w, xa_ref[...], (((1,), (1,)), ((), ())),
            preferred_element_type=jnp.float32,
        ),
        jax.lax.dot_general(
            w, xb_ref[...], (((1,), (1,)), ((), ())),
            preferred_element_type=jnp.float32,
        ),
    ], axis=1)
    m = jnp.max(scores, axis=0, keepdims=True)
    e = jnp.exp(scores - m)
    s = jnp.sum(e, axis=0, keepdims=True)
    probs = e / s
    probs_ref[...] = probs

    v1 = jnp.max(probs, axis=0, keepdims=True)
    i1 = jnp.argmax(probs, axis=0).reshape(1, -1)
    row = jax.lax.broadcasted_iota(jnp.int32, probs.shape, 0)
    masked = jnp.where(row == i1, -jnp.inf, probs)
    v2 = jnp.max(masked, axis=0, keepdims=True)
    i2 = jnp.argmax(masked, axis=0).reshape(1, -1)
    denom = v1 + v2 + 1e-9
    tv_ref[...] = jnp.concatenate([v1 / denom, v2 / denom], axis=0)
    ti_ref[...] = jnp.concatenate([i1, i2], axis=0).astype(jnp.int32)


def kernel(x, weight):
    n_tok, dim = x.shape
    n_exp = weight.shape[0]
    t2 = BLOCK_T // 2
    grid = (n_tok // BLOCK_T,)
    probs_t, tv_t, ti_t = pl.pallas_call(
        _gate_kernel,
        grid=grid,
        in_specs=[
            pl.BlockSpec((t2, dim), lambda i: (2 * i, 0)),
            pl.BlockSpec((t2, dim), lambda i: (2 * i + 1, 0)),
            pl.BlockSpec((n_exp, dim), lambda i: (0, 0)),
        ],
        out_specs=[
            pl.BlockSpec((n_exp, BLOCK_T), lambda i: (0, i)),
            pl.BlockSpec((TOP_K, BLOCK_T), lambda i: (0, i)),
            pl.BlockSpec((TOP_K, BLOCK_T), lambda i: (0, i)),
        ],
        out_shape=[
            jax.ShapeDtypeStruct((n_exp, n_tok), jnp.float32),
            jax.ShapeDtypeStruct((TOP_K, n_tok), jnp.float32),
            jax.ShapeDtypeStruct((TOP_K, n_tok), jnp.int32),
        ],
    )(x, x, weight)
    return tv_t.T, ti_t.T, probs_t.T
